# baseline (speedup 1.0000x reference)
import functools

import jax
import jax.numpy as jnp
from jax import lax
from jax.experimental import pallas as pl
from jax.experimental.pallas import tpu as pltpu
from jax.experimental.pallas import tpu_sc as plsc

NC = 2
NS = 16
NW = NC * NS
WIN = 128
NBUF = 2
CHW = 16
ROW_BLK = 400


def _msg_body(x_ref, wg_ref, we_ref, o_ref):
    x = x_ref[...]
    g = jnp.dot(x, wg_ref[...], preferred_element_type=jnp.float32)
    e = jnp.dot(x, we_ref[...], preferred_element_type=jnp.float32)
    o_ref[...] = jax.nn.sigmoid(g) * e


def _messages(x, wg_t, we_t):
    n, d = x.shape
    return pl.pallas_call(
        _msg_body,
        grid=(n // ROW_BLK,),
        in_specs=[
            pl.BlockSpec((ROW_BLK, d), lambda i: (i, 0)),
            pl.BlockSpec((d, d), lambda i: (0, 0)),
            pl.BlockSpec((d, d), lambda i: (0, 0)),
        ],
        out_specs=pl.BlockSpec((ROW_BLK, d), lambda i: (i, 0)),
        out_shape=jax.ShapeDtypeStruct((n, d), jnp.float32),
    )(x, wg_t, we_t)


def _combine_body(x_ref, p_ref, o_ref):
    o_ref[...] = x_ref[...] + p_ref[0] + p_ref[1]


def _combine(x, p):
    n, d = x.shape
    return pl.pallas_call(
        _combine_body,
        grid=(n // ROW_BLK,),
        in_specs=[
            pl.BlockSpec((ROW_BLK, d), lambda i: (i, 0)),
            pl.BlockSpec((2, ROW_BLK, d), lambda i: (0, i, 0)),
        ],
        out_specs=pl.BlockSpec((ROW_BLK, d), lambda i: (i, 0)),
        out_shape=jax.ShapeDtypeStruct((n, d), jnp.float32),
    )(x, p)


def _sc_partials(msg, src3, tgt3, n_nodes, acc_rows, a_win, b_win):
    kwin = src3.shape[1]
    d = msg.shape[1]
    rows_per = acc_rows // NS
    zch = acc_rows // NS // WIN
    mesh = plsc.VectorSubcoreMesh(core_axis_name="c", subcore_axis_name="s")

    @functools.partial(
        pl.kernel,
        mesh=mesh,
        out_type=jax.ShapeDtypeStruct((NC, acc_rows, d), jnp.float32),
        scratch_types=[
            pltpu.VMEM((2, CHW, WIN), jnp.int32),
            pltpu.VMEM((2, CHW, WIN), jnp.int32),
            pltpu.VMEM((NBUF, WIN, d), jnp.float32),
            pltpu.VMEM_SHARED((acc_rows, d), jnp.float32),
            pltpu.SemaphoreType.DMA,
            pltpu.SemaphoreType.DMA,
            pltpu.SemaphoreType.DMA,
        ],
    )
    def k_fn(msg_hbm, src_hbm, tgt_hbm, out_hbm, src_v, tgt_v, bufs, acc,
             sem0, sem1, isem):
        cid = lax.axis_index("c")
        sid = lax.axis_index("s")
        wid = cid * NS + sid
        sems = (sem0, sem1)
        buf0 = bufs.at[0]

        with jax.named_scope("zero_fill"):
            @pl.loop(0, WIN)
            def _(i):
                @pl.loop(0, d, step=16)
                def _(jj):
                    bufs[0, i, pl.ds(jj, 16)] = jnp.zeros((16,), jnp.float32)

            zbase = sid * (acc_rows // NS)

            @pl.loop(0, zch)
            def _(i):
                pltpu.sync_copy(buf0, acc.at[pl.ds(zbase + i * WIN, WIN)])

            plsc.subcore_barrier()

        my_src = src_hbm.at[wid]
        my_tgt = tgt_hbm.at[wid]
        with jax.named_scope("idx_stage0"):
            pltpu.sync_copy(my_src.at[pl.ds(0, CHW)], src_v.at[0])
            pltpu.sync_copy(my_tgt.at[pl.ds(0, CHW)], tgt_v.at[0])

        nch = jnp.where(cid == 0, a_win, b_win) // CHW

        with jax.named_scope("gather_scatter"):
            @pl.loop(0, nch)
            def _(c):
                par = c % 2
                sv = src_v.at[par]
                tv = tgt_v.at[par]

                @pl.when(c + 1 < nch)
                def _():
                    pltpu.async_copy(
                        my_src.at[pl.ds((c + 1) * CHW, CHW)],
                        src_v.at[1 - par], isem
                    )
                    pltpu.async_copy(
                        my_tgt.at[pl.ds((c + 1) * CHW, CHW)],
                        tgt_v.at[1 - par], isem
                    )

                for b in range(NBUF):
                    pltpu.async_copy(msg_hbm.at[tv.at[b]], bufs.at[b], sems[b])

                @pl.loop(0, CHW, step=NBUF)
                def _(j):
                    for b in range(NBUF):
                        pltpu.make_async_copy(
                            msg_hbm.at[tv.at[0]], bufs.at[b], sems[b]
                        ).wait()
                        pltpu.sync_copy(bufs.at[b], acc.at[sv.at[j + b]],
                                        add=True)

                        @pl.when(j + b + NBUF < CHW)
                        def _():
                            pltpu.async_copy(
                                msg_hbm.at[tv.at[j + b + NBUF]],
                                bufs.at[b], sems[b]
                            )

                @pl.when(c + 1 < nch)
                def _():
                    pltpu.make_async_copy(
                        my_src.at[pl.ds(0, CHW)], src_v.at[1 - par], isem
                    ).wait()
                    pltpu.make_async_copy(
                        my_tgt.at[pl.ds(0, CHW)], tgt_v.at[1 - par], isem
                    ).wait()

        plsc.subcore_barrier()

        with jax.named_scope("copy_out"):
            rbase = sid * rows_per
            pltpu.sync_copy(
                acc.at[pl.ds(rbase, rows_per)],
                out_hbm.at[cid].at[pl.ds(rbase, rows_per)],
            )

    return k_fn(msg, src3, tgt3)


def kernel(input, edge_sources, edge_targets, W):
    x = input
    n, d = x.shape
    dout = W.shape[0] // 2
    wg_t = W[:dout].T
    we_t = W[dout:].T
    msg = _messages(x, wg_t, we_t)

    e = edge_sources.shape[0]
    tw = -(-e // WIN)
    a_win = -(-(tw * 4) // (5 * NS * CHW)) * CHW
    b_win = max(-(-(tw - NS * a_win) // (NS * CHW)) * CHW, CHW)
    kmax = max(a_win, b_win)
    epad = NS * (a_win + b_win) * WIN
    acc_rows = -(-(n + 1) // (NS * WIN)) * (NS * WIN)
    src = jnp.full((epad,), n, jnp.int32).at[:e].set(edge_sources.astype(jnp.int32))
    tgt = jnp.zeros((epad,), jnp.int32).at[:e].set(edge_targets.astype(jnp.int32))

    def layout(idx):
        c0 = idx[: NS * a_win * WIN].reshape(NS, a_win, WIN)
        c1 = idx[NS * a_win * WIN :].reshape(NS, b_win, WIN)
        c0 = jnp.pad(c0, ((0, 0), (0, kmax - a_win), (0, 0)))
        c1 = jnp.pad(c1, ((0, 0), (0, kmax - b_win), (0, 0)))
        return jnp.concatenate([c0, c1], axis=0)

    partial = _sc_partials(msg, layout(src), layout(tgt), n, acc_rows,
                           a_win, b_win)
    return _combine(x, partial)


# speedup vs baseline: 2.6565x; 2.6565x over previous
import functools

import jax
import jax.numpy as jnp
from jax import lax
from jax.experimental import pallas as pl
from jax.experimental.pallas import tpu as pltpu
from jax.experimental.pallas import tpu_sc as plsc

NC = 2
NS = 16
NW = NC * NS
WIN = 128
NBUF = 2
CHW = 16
ROW_BLK = 400


def _msg_body(x_ref, wg_ref, we_ref, o_ref):
    x = x_ref[...]
    g = jnp.dot(x, wg_ref[...], preferred_element_type=jnp.float32)
    e = jnp.dot(x, we_ref[...], preferred_element_type=jnp.float32)
    o_ref[...] = jax.nn.sigmoid(g) * e


def _messages(x, wg_t, we_t):
    n, d = x.shape
    return pl.pallas_call(
        _msg_body,
        grid=(n // ROW_BLK,),
        in_specs=[
            pl.BlockSpec((ROW_BLK, d), lambda i: (i, 0)),
            pl.BlockSpec((d, d), lambda i: (0, 0)),
            pl.BlockSpec((d, d), lambda i: (0, 0)),
        ],
        out_specs=pl.BlockSpec((ROW_BLK, d), lambda i: (i, 0)),
        out_shape=jax.ShapeDtypeStruct((n, d), jnp.float32),
    )(x, wg_t, we_t)


def _combine_body(x_ref, p_ref, o_ref):
    o_ref[...] = x_ref[...] + p_ref[0] + p_ref[1]


def _combine(x, p):
    n, d = x.shape
    return pl.pallas_call(
        _combine_body,
        grid=(n // ROW_BLK,),
        in_specs=[
            pl.BlockSpec((ROW_BLK, d), lambda i: (i, 0)),
            pl.BlockSpec((2, ROW_BLK, d), lambda i: (0, i, 0)),
        ],
        out_specs=pl.BlockSpec((ROW_BLK, d), lambda i: (i, 0)),
        out_shape=jax.ShapeDtypeStruct((n, d), jnp.float32),
    )(x, p)


def _sc_partials(msg, src3, tgt3, n_nodes, acc_rows, a_win, b_win):
    kwin = src3.shape[1]
    d = msg.shape[1]
    rows_per = acc_rows // NS
    zch = acc_rows // NS // WIN
    mesh = plsc.VectorSubcoreMesh(core_axis_name="c", subcore_axis_name="s")

    @functools.partial(
        pl.kernel,
        mesh=mesh,
        out_type=jax.ShapeDtypeStruct((NC, acc_rows, d), jnp.float32),
        scratch_types=[
            pltpu.VMEM((2, CHW, WIN), jnp.int32),
            pltpu.VMEM((2, CHW, WIN), jnp.int32),
            pltpu.VMEM((NBUF, WIN, d), jnp.float32),
            pltpu.VMEM_SHARED((acc_rows, d), jnp.float32),
            pltpu.SemaphoreType.DMA,
            pltpu.SemaphoreType.DMA,
            pltpu.SemaphoreType.DMA,
        ],
    )
    def k_fn(msg_hbm, src_hbm, tgt_hbm, out_hbm, src_v, tgt_v, bufs, acc,
             sem0, sem1, isem):
        cid = lax.axis_index("c")
        sid = lax.axis_index("s")
        wid = cid * NS + sid
        sems = (sem0, sem1)
        buf0 = bufs.at[0]

        with jax.named_scope("zero_fill"):
            @pl.loop(0, WIN)
            def _(i):
                @pl.loop(0, d, step=16)
                def _(jj):
                    bufs[0, i, pl.ds(jj, 16)] = jnp.zeros((16,), jnp.float32)

            zbase = sid * (acc_rows // NS)

            @pl.loop(0, zch)
            def _(i):
                pltpu.sync_copy(buf0, acc.at[pl.ds(zbase + i * WIN, WIN)])

            plsc.subcore_barrier()

        my_src = src_hbm.at[wid]
        my_tgt = tgt_hbm.at[wid]
        with jax.named_scope("idx_stage0"):
            pltpu.sync_copy(my_src.at[pl.ds(0, CHW)], src_v.at[0])
            pltpu.sync_copy(my_tgt.at[pl.ds(0, CHW)], tgt_v.at[0])

        nch = jnp.where(cid == 0, a_win, b_win) // CHW

        with jax.named_scope("gather_scatter"):
            @pl.loop(0, nch)
            def _(c):
                par = c % 2
                sv = src_v.at[par]
                tv = tgt_v.at[par]

                @pl.when(c + 1 < nch)
                def _():
                    pltpu.async_copy(
                        my_src.at[pl.ds((c + 1) * CHW, CHW)],
                        src_v.at[1 - par], isem
                    )
                    pltpu.async_copy(
                        my_tgt.at[pl.ds((c + 1) * CHW, CHW)],
                        tgt_v.at[1 - par], isem
                    )

                for b in range(NBUF):
                    pltpu.async_copy(msg_hbm.at[tv.at[b]], bufs.at[b], sems[b])

                @pl.loop(0, CHW, step=NBUF)
                def _(j):
                    for b in range(NBUF):
                        pltpu.make_async_copy(
                            msg_hbm.at[tv.at[0]], bufs.at[b], sems[b]
                        ).wait()
                        pltpu.sync_copy(bufs.at[b], acc.at[sv.at[j + b]],
                                        add=True)

                        @pl.when(j + b + NBUF < CHW)
                        def _():
                            pltpu.async_copy(
                                msg_hbm.at[tv.at[j + b + NBUF]],
                                bufs.at[b], sems[b]
                            )

                @pl.when(c + 1 < nch)
                def _():
                    pltpu.make_async_copy(
                        my_src.at[pl.ds(0, CHW)], src_v.at[1 - par], isem
                    ).wait()
                    pltpu.make_async_copy(
                        my_tgt.at[pl.ds(0, CHW)], tgt_v.at[1 - par], isem
                    ).wait()

        plsc.subcore_barrier()

        with jax.named_scope("copy_out"):
            rbase = sid * rows_per
            pltpu.sync_copy(
                acc.at[pl.ds(rbase, rows_per)],
                out_hbm.at[cid].at[pl.ds(rbase, rows_per)],
            )

    return k_fn(msg, src3, tgt3)


def kernel(input, edge_sources, edge_targets, W):
    x = input
    n, d = x.shape
    dout = W.shape[0] // 2
    wg_t = W[:dout].T
    we_t = W[dout:].T
    msg = _messages(x, wg_t, we_t)

    e = edge_sources.shape[0]
    kwin = -(-e // (NW * WIN))
    kwin = -(-kwin // CHW) * CHW
    epad = NW * kwin * WIN
    acc_rows = -(-(n + WIN) // (NS * WIN)) * (NS * WIN)
    lanes = jax.lax.iota(jnp.int32, epad)
    src = (n + lanes % WIN).at[:e].set(edge_sources.astype(jnp.int32))
    tgt = (lanes % WIN).at[:e].set(edge_targets.astype(jnp.int32))
    partial = _sc_partials(
        msg,
        src.reshape(NW, kwin, WIN),
        tgt.reshape(NW, kwin, WIN),
        n,
        acc_rows,
        kwin,
        kwin,
    )
    return _combine(x, partial)


# speedup vs baseline: 3.2468x; 1.2222x over previous
import functools

import jax
import jax.numpy as jnp
from jax import lax
from jax.experimental import pallas as pl
from jax.experimental.pallas import tpu as pltpu
from jax.experimental.pallas import tpu_sc as plsc

NC = 2
NS = 16
NW = NC * NS
WIN = 128
NBUF = 2
CHW = 16
ROW_BLK = 2000


def _msg_body(x_ref, wg_ref, we_ref, o_ref):
    x = x_ref[...]
    g = jnp.dot(x, wg_ref[...], preferred_element_type=jnp.float32)
    e = jnp.dot(x, we_ref[...], preferred_element_type=jnp.float32)
    o_ref[...] = jax.nn.sigmoid(g) * e


def _messages(x, wg_t, we_t):
    n, d = x.shape
    return pl.pallas_call(
        _msg_body,
        grid=(n // ROW_BLK,),
        in_specs=[
            pl.BlockSpec((ROW_BLK, d), lambda i: (i, 0)),
            pl.BlockSpec((d, d), lambda i: (0, 0)),
            pl.BlockSpec((d, d), lambda i: (0, 0)),
        ],
        out_specs=pl.BlockSpec((ROW_BLK, d), lambda i: (i, 0)),
        out_shape=jax.ShapeDtypeStruct((n, d), jnp.float32),
    )(x, wg_t, we_t)


def _combine_body(x_ref, p_ref, o_ref):
    o_ref[...] = x_ref[...] + p_ref[0] + p_ref[1]


def _combine(x, p):
    n, d = x.shape
    return pl.pallas_call(
        _combine_body,
        grid=(n // ROW_BLK,),
        in_specs=[
            pl.BlockSpec((ROW_BLK, d), lambda i: (i, 0)),
            pl.BlockSpec((2, ROW_BLK, d), lambda i: (0, i, 0)),
        ],
        out_specs=pl.BlockSpec((ROW_BLK, d), lambda i: (i, 0)),
        out_shape=jax.ShapeDtypeStruct((n, d), jnp.float32),
    )(x, p)


def _sc_partials(msg, src3, tgt3, n_nodes, acc_rows, a_win, b_win):
    kwin = src3.shape[1]
    d = msg.shape[1]
    rows_per = acc_rows // NS
    zch = acc_rows // NS // WIN
    mesh = plsc.VectorSubcoreMesh(core_axis_name="c", subcore_axis_name="s")

    @functools.partial(
        pl.kernel,
        mesh=mesh,
        out_type=jax.ShapeDtypeStruct((NC, acc_rows, d), jnp.float32),
        scratch_types=[
            pltpu.VMEM((2, CHW, WIN), jnp.int32),
            pltpu.VMEM((2, CHW, WIN), jnp.int32),
            pltpu.VMEM((NBUF, WIN, d), jnp.float32),
            pltpu.VMEM_SHARED((acc_rows, d), jnp.float32),
            pltpu.SemaphoreType.DMA,
            pltpu.SemaphoreType.DMA,
            pltpu.SemaphoreType.DMA,
        ],
    )
    def k_fn(msg_hbm, src_hbm, tgt_hbm, out_hbm, src_v, tgt_v, bufs, acc,
             sem0, sem1, isem):
        cid = lax.axis_index("c")
        sid = lax.axis_index("s")
        wid = cid * NS + sid
        sems = (sem0, sem1)
        buf0 = bufs.at[0]

        with jax.named_scope("zero_fill"):
            @pl.loop(0, WIN)
            def _(i):
                @pl.loop(0, d, step=16)
                def _(jj):
                    bufs[0, i, pl.ds(jj, 16)] = jnp.zeros((16,), jnp.float32)

            zbase = sid * (acc_rows // NS)

            @pl.loop(0, zch)
            def _(i):
                pltpu.sync_copy(buf0, acc.at[pl.ds(zbase + i * WIN, WIN)])

            plsc.subcore_barrier()

        my_src = src_hbm.at[wid]
        my_tgt = tgt_hbm.at[wid]
        with jax.named_scope("idx_stage0"):
            pltpu.sync_copy(my_src.at[pl.ds(0, CHW)], src_v.at[0])
            pltpu.sync_copy(my_tgt.at[pl.ds(0, CHW)], tgt_v.at[0])

        nch = jnp.where(cid == 0, a_win, b_win) // CHW

        with jax.named_scope("gather_scatter"):
            @pl.loop(0, nch)
            def _(c):
                par = c % 2
                sv = src_v.at[par]
                tv = tgt_v.at[par]

                @pl.when(c + 1 < nch)
                def _():
                    pltpu.async_copy(
                        my_src.at[pl.ds((c + 1) * CHW, CHW)],
                        src_v.at[1 - par], isem
                    )
                    pltpu.async_copy(
                        my_tgt.at[pl.ds((c + 1) * CHW, CHW)],
                        tgt_v.at[1 - par], isem
                    )

                for b in range(NBUF):
                    pltpu.async_copy(msg_hbm.at[tv.at[b]], bufs.at[b], sems[b])

                @pl.loop(0, CHW, step=NBUF)
                def _(j):
                    for b in range(NBUF):
                        pltpu.make_async_copy(
                            msg_hbm.at[tv.at[0]], bufs.at[b], sems[b]
                        ).wait()
                        pltpu.sync_copy(bufs.at[b], acc.at[sv.at[j + b]],
                                        add=True)

                        @pl.when(j + b + NBUF < CHW)
                        def _():
                            pltpu.async_copy(
                                msg_hbm.at[tv.at[j + b + NBUF]],
                                bufs.at[b], sems[b]
                            )

                @pl.when(c + 1 < nch)
                def _():
                    pltpu.make_async_copy(
                        my_src.at[pl.ds(0, CHW)], src_v.at[1 - par], isem
                    ).wait()
                    pltpu.make_async_copy(
                        my_tgt.at[pl.ds(0, CHW)], tgt_v.at[1 - par], isem
                    ).wait()

        plsc.subcore_barrier()

        with jax.named_scope("copy_out"):
            rbase = sid * rows_per
            pltpu.sync_copy(
                acc.at[pl.ds(rbase, rows_per)],
                out_hbm.at[cid].at[pl.ds(rbase, rows_per)],
            )

    return k_fn(msg, src3, tgt3)


def kernel(input, edge_sources, edge_targets, W):
    x = input
    n, d = x.shape
    dout = W.shape[0] // 2
    wg_t = W[:dout].T
    we_t = W[dout:].T
    msg = _messages(x, wg_t, we_t)

    e = edge_sources.shape[0]
    kwin = -(-e // (NW * WIN))
    kwin = -(-kwin // CHW) * CHW
    epad = NW * kwin * WIN
    acc_rows = -(-(n + WIN) // (NS * WIN)) * (NS * WIN)
    lanes = jax.lax.iota(jnp.int32, epad - e) % WIN
    src = jnp.concatenate([edge_sources.astype(jnp.int32), n + lanes])
    tgt = jnp.concatenate([edge_targets.astype(jnp.int32), lanes])
    partial = _sc_partials(
        msg,
        src.reshape(NW, kwin, WIN),
        tgt.reshape(NW, kwin, WIN),
        n,
        acc_rows,
        kwin,
        kwin,
    )
    return _combine(x, partial)


# speedup vs baseline: 3.3029x; 1.0173x over previous
import functools

import jax
import jax.numpy as jnp
from jax import lax
from jax.experimental import pallas as pl
from jax.experimental.pallas import tpu as pltpu
from jax.experimental.pallas import tpu_sc as plsc

NC = 2
NS = 16
NW = NC * NS
WIN = 64
NBUF = 4
CHW = 16
ROW_BLK = 2000


def _msg_body(x_ref, wg_ref, we_ref, o_ref):
    x = x_ref[...]
    g = jnp.dot(x, wg_ref[...], preferred_element_type=jnp.float32)
    e = jnp.dot(x, we_ref[...], preferred_element_type=jnp.float32)
    o_ref[...] = jax.nn.sigmoid(g) * e


def _messages(x, wg_t, we_t):
    n, d = x.shape
    return pl.pallas_call(
        _msg_body,
        grid=(n // ROW_BLK,),
        in_specs=[
            pl.BlockSpec((ROW_BLK, d), lambda i: (i, 0)),
            pl.BlockSpec((d, d), lambda i: (0, 0)),
            pl.BlockSpec((d, d), lambda i: (0, 0)),
        ],
        out_specs=pl.BlockSpec((ROW_BLK, d), lambda i: (i, 0)),
        out_shape=jax.ShapeDtypeStruct((n, d), jnp.float32),
    )(x, wg_t, we_t)


def _combine_body(x_ref, p_ref, o_ref):
    o_ref[...] = x_ref[...] + p_ref[0] + p_ref[1]


def _combine(x, p):
    n, d = x.shape
    return pl.pallas_call(
        _combine_body,
        grid=(n // ROW_BLK,),
        in_specs=[
            pl.BlockSpec((ROW_BLK, d), lambda i: (i, 0)),
            pl.BlockSpec((2, ROW_BLK, d), lambda i: (0, i, 0)),
        ],
        out_specs=pl.BlockSpec((ROW_BLK, d), lambda i: (i, 0)),
        out_shape=jax.ShapeDtypeStruct((n, d), jnp.float32),
    )(x, p)


def _sc_partials(msg, src3, tgt3, n_nodes, acc_rows, a_win, b_win):
    kwin = src3.shape[1]
    d = msg.shape[1]
    rows_per = acc_rows // NS
    zch = acc_rows // NS // WIN
    mesh = plsc.VectorSubcoreMesh(core_axis_name="c", subcore_axis_name="s")

    @functools.partial(
        pl.kernel,
        mesh=mesh,
        out_type=jax.ShapeDtypeStruct((NC, acc_rows, d), jnp.float32),
        scratch_types=[
            pltpu.VMEM((2, CHW, WIN), jnp.int32),
            pltpu.VMEM((2, CHW, WIN), jnp.int32),
            pltpu.VMEM((NBUF, WIN, d), jnp.float32),
            pltpu.VMEM_SHARED((acc_rows, d), jnp.float32),
            pltpu.SemaphoreType.DMA,
            pltpu.SemaphoreType.DMA,
            pltpu.SemaphoreType.DMA,
            pltpu.SemaphoreType.DMA,
            pltpu.SemaphoreType.DMA,
        ],
    )
    def k_fn(msg_hbm, src_hbm, tgt_hbm, out_hbm, src_v, tgt_v, bufs, acc,
             sem0, sem1, sem2, sem3, isem):
        cid = lax.axis_index("c")
        sid = lax.axis_index("s")
        wid = cid * NS + sid
        sems = (sem0, sem1, sem2, sem3)
        buf0 = bufs.at[0]

        with jax.named_scope("zero_fill"):
            @pl.loop(0, WIN)
            def _(i):
                @pl.loop(0, d, step=16)
                def _(jj):
                    bufs[0, i, pl.ds(jj, 16)] = jnp.zeros((16,), jnp.float32)

            zbase = sid * (acc_rows // NS)

            @pl.loop(0, zch)
            def _(i):
                pltpu.sync_copy(buf0, acc.at[pl.ds(zbase + i * WIN, WIN)])

            plsc.subcore_barrier()

        my_src = src_hbm.at[wid]
        my_tgt = tgt_hbm.at[wid]
        with jax.named_scope("idx_stage0"):
            pltpu.sync_copy(my_src.at[pl.ds(0, CHW)], src_v.at[0])
            pltpu.sync_copy(my_tgt.at[pl.ds(0, CHW)], tgt_v.at[0])

        nch = jnp.where(cid == 0, a_win, b_win) // CHW

        with jax.named_scope("gather_scatter"):
            @pl.loop(0, nch)
            def _(c):
                par = c % 2
                sv = src_v.at[par]
                tv = tgt_v.at[par]

                @pl.when(c + 1 < nch)
                def _():
                    pltpu.async_copy(
                        my_src.at[pl.ds((c + 1) * CHW, CHW)],
                        src_v.at[1 - par], isem
                    )
                    pltpu.async_copy(
                        my_tgt.at[pl.ds((c + 1) * CHW, CHW)],
                        tgt_v.at[1 - par], isem
                    )

                for b in range(NBUF):
                    pltpu.async_copy(msg_hbm.at[tv.at[b]], bufs.at[b], sems[b])

                @pl.loop(0, CHW, step=NBUF)
                def _(j):
                    for b in range(NBUF):
                        pltpu.make_async_copy(
                            msg_hbm.at[tv.at[0]], bufs.at[b], sems[b]
                        ).wait()
                        pltpu.sync_copy(bufs.at[b], acc.at[sv.at[j + b]],
                                        add=True)

                        @pl.when(j + b + NBUF < CHW)
                        def _():
                            pltpu.async_copy(
                                msg_hbm.at[tv.at[j + b + NBUF]],
                                bufs.at[b], sems[b]
                            )

                @pl.when(c + 1 < nch)
                def _():
                    pltpu.make_async_copy(
                        my_src.at[pl.ds(0, CHW)], src_v.at[1 - par], isem
                    ).wait()
                    pltpu.make_async_copy(
                        my_tgt.at[pl.ds(0, CHW)], tgt_v.at[1 - par], isem
                    ).wait()

        plsc.subcore_barrier()

        with jax.named_scope("copy_out"):
            rbase = sid * rows_per
            pltpu.sync_copy(
                acc.at[pl.ds(rbase, rows_per)],
                out_hbm.at[cid].at[pl.ds(rbase, rows_per)],
            )

    return k_fn(msg, src3, tgt3)


def kernel(input, edge_sources, edge_targets, W):
    x = input
    n, d = x.shape
    dout = W.shape[0] // 2
    wg_t = W[:dout].T
    we_t = W[dout:].T
    msg = _messages(x, wg_t, we_t)

    e = edge_sources.shape[0]
    kwin = -(-e // (NW * WIN))
    kwin = -(-kwin // CHW) * CHW
    epad = NW * kwin * WIN
    acc_rows = -(-(n + WIN) // (NS * WIN)) * (NS * WIN)
    lanes = jax.lax.iota(jnp.int32, epad - e) % WIN
    src = jnp.concatenate([edge_sources.astype(jnp.int32), n + lanes])
    tgt = jnp.concatenate([edge_targets.astype(jnp.int32), lanes])
    partial = _sc_partials(
        msg,
        src.reshape(NW, kwin, WIN),
        tgt.reshape(NW, kwin, WIN),
        n,
        acc_rows,
        kwin,
        kwin,
    )
    return _combine(x, partial)
